# baseline (speedup 1.0000x reference)
import functools

import jax
import jax.numpy as jnp
from jax import lax
from jax.experimental import pallas as pl
from jax.experimental.pallas import tpu as pltpu
from jax.experimental.pallas import tpu_sc as plsc

B = 16384
D = 256
L = 16
NC = 2
NS = 16
NW = NC * NS
BPW = B // NW
Q = 4
NQ = BPW // Q
NPAT = 1 << Q

_mesh = plsc.VectorSubcoreMesh(core_axis_name="c", subcore_axis_name="s")


@functools.partial(
    pl.kernel,
    mesh=_mesh,
    out_type=jax.ShapeDtypeStruct((B, D), jnp.float32),
    scratch_types=[
        pltpu.VMEM((BPW + L,), jnp.int32),
        pltpu.VMEM((2, D), jnp.float32),
        pltpu.VMEM((NPAT * Q, D), jnp.float32),
        pltpu.VMEM_SHARED((NPAT * Q, D), jnp.float32),
        pltpu.SemaphoreType.DMA,
        pltpu.SemaphoreType.DMA,
    ],
)
def _lookup(idx_hbm, table_hbm, out_hbm, idx_v, table_v, pat_v, pat_sh,
            sem, sem2):
    wid = lax.axis_index("s") * NC + lax.axis_index("c")
    base = wid * BPW
    pltpu.sync_copy(table_hbm, table_v)
    pltpu.sync_copy(idx_hbm.at[wid], idx_v.at[pl.ds(0, BPW)])
    w0 = [table_v[0, pl.ds(c * L, L)] for c in range(D // L)]
    w1 = [table_v[1, pl.ds(c * L, L)] for c in range(D // L)]

    def build(k, carry):
        h = k & (Q - 1)
        p = k >> 2
        bit = lax.shift_right_logical(p, (Q - 1) - h) & 1
        take1 = bit != 0
        for c in range(D // L):
            pat_v[k, pl.ds(c * L, L)] = jnp.where(take1, w1[c], w0[c])
        return carry
    lax.fori_loop(0, NPAT * Q, build, 0)

    @pl.when(lax.axis_index("s") == 0)
    def _():
        pltpu.sync_copy(pat_v, pat_sh)
    plsc.subcore_barrier()

    def grp(g, carry):
        v = idx_v[pl.ds(L * g, L)]
        for j in range(L // Q):
            q = ((v[Q * j] * 2 + v[Q * j + 1]) * 2 + v[Q * j + 2]) * 2 \
                + v[Q * j + 3]
            src, s = (pat_v, sem) if j % 2 == 0 else (pat_sh, sem2)
            pltpu.async_copy(
                src.at[pl.ds(Q * q, Q)],
                out_hbm.at[pl.ds(base + L * g + Q * j, Q)],
                s)
        return carry
    lax.fori_loop(0, BPW // L, grp, 0)

    half = out_hbm.at[pl.ds(base, BPW // 2)]
    pltpu.make_async_copy(half, half, sem).wait()
    pltpu.make_async_copy(half, half, sem2).wait()


def kernel(harmful, emb_weight):
    idx = jnp.reshape(harmful.astype(jnp.int32), (NW, BPW))
    return _lookup(idx, emb_weight)


# speedup vs baseline: 1.0401x; 1.0401x over previous
import functools

import jax
import jax.numpy as jnp
from jax import lax
from jax.experimental import pallas as pl
from jax.experimental.pallas import tpu as pltpu
from jax.experimental.pallas import tpu_sc as plsc

B = 16384
D = 256
L = 16
NC = 2
NS = 16
NW = NC * NS
BPW = B // NW
Q = 4
NQ = BPW // Q
NPAT = 1 << Q

_mesh = plsc.VectorSubcoreMesh(core_axis_name="c", subcore_axis_name="s")


@functools.partial(
    pl.kernel,
    mesh=_mesh,
    out_type=jax.ShapeDtypeStruct((B, D), jnp.float32),
    scratch_types=[
        pltpu.VMEM((BPW + L,), jnp.int32),
        pltpu.VMEM((2, D), jnp.float32),
        pltpu.VMEM((NPAT * Q, D), jnp.float32),
        pltpu.SemaphoreType.DMA,
    ],
)
def _lookup(idx_hbm, table_hbm, out_hbm, idx_v, table_v, pat_v, sem):
    wid = lax.axis_index("s") * NC + lax.axis_index("c")
    base = wid * BPW
    pltpu.sync_copy(table_hbm, table_v)
    pltpu.sync_copy(idx_hbm.at[wid], idx_v.at[pl.ds(0, BPW)])
    w0 = [table_v[0, pl.ds(c * L, L)] for c in range(D // L)]
    w1 = [table_v[1, pl.ds(c * L, L)] for c in range(D // L)]

    def build(k, carry):
        h = k & (Q - 1)
        p = k >> 2
        bit = lax.shift_right_logical(p, (Q - 1) - h) & 1
        take1 = bit != 0
        for c in range(D // L):
            pat_v[k, pl.ds(c * L, L)] = jnp.where(take1, w1[c], w0[c])
        return carry
    lax.fori_loop(0, NPAT * Q, build, 0)

    def grp(g, carry):
        v = idx_v[pl.ds(L * g, L)]
        for j in range(L // Q):
            q = ((v[Q * j] * 2 + v[Q * j + 1]) * 2 + v[Q * j + 2]) * 2 \
                + v[Q * j + 3]
            pltpu.async_copy(
                pat_v.at[pl.ds(Q * q, Q)],
                out_hbm.at[pl.ds(base + L * g + Q * j, Q)],
                sem)
        return carry
    lax.fori_loop(0, BPW // L, grp, 0)

    my_out = out_hbm.at[pl.ds(base, BPW)]
    pltpu.make_async_copy(my_out, my_out, sem).wait()


def kernel(harmful, emb_weight):
    idx = jnp.reshape(harmful.astype(jnp.int32), (NW, BPW))
    return _lookup(idx, emb_weight)


# speedup vs baseline: 1.0524x; 1.0118x over previous
import functools

import jax
import jax.numpy as jnp
from jax import lax
from jax.experimental import pallas as pl
from jax.experimental.pallas import tpu as pltpu
from jax.experimental.pallas import tpu_sc as plsc

B = 16384
D = 256
L = 16
NC = 2
NS = 16
NW = NC * NS
BPW = B // NW
Q = 2
NQ = BPW // Q
NPAT = 1 << Q

_mesh = plsc.VectorSubcoreMesh(core_axis_name="c", subcore_axis_name="s")


@functools.partial(
    pl.kernel,
    mesh=_mesh,
    out_type=jax.ShapeDtypeStruct((B, D), jnp.float32),
    scratch_types=[
        pltpu.VMEM((BPW + L,), jnp.int32),
        pltpu.VMEM((2, D), jnp.float32),
        pltpu.VMEM((NPAT * Q, D), jnp.float32),
        pltpu.SemaphoreType.DMA,
    ],
)
def _lookup(idx_hbm, table_hbm, out_hbm, idx_v, table_v, pat_v, sem):
    wid = lax.axis_index("s") * NC + lax.axis_index("c")
    base = wid * BPW
    pltpu.sync_copy(table_hbm, table_v)
    pltpu.sync_copy(idx_hbm.at[wid], idx_v.at[pl.ds(0, BPW)])
    w0 = [table_v[0, pl.ds(c * L, L)] for c in range(D // L)]
    w1 = [table_v[1, pl.ds(c * L, L)] for c in range(D // L)]

    def build(k, carry):
        h = k & (Q - 1)
        p = k // Q
        bit = lax.shift_right_logical(p, (Q - 1) - h) & 1
        take1 = bit != 0
        for c in range(D // L):
            pat_v[k, pl.ds(c * L, L)] = jnp.where(take1, w1[c], w0[c])
        return carry
    lax.fori_loop(0, NPAT * Q, build, 0)

    def grp(g, carry):
        v = idx_v[pl.ds(L * g, L)]
        for j in range(L // Q):
            q = v[Q * j]
            for t in range(1, Q):
                q = q * 2 + v[Q * j + t]
            pltpu.async_copy(
                pat_v.at[pl.ds(Q * q, Q)],
                out_hbm.at[pl.ds(base + L * g + Q * j, Q)],
                sem)
        return carry
    lax.fori_loop(0, BPW // L, grp, 0)

    my_out = out_hbm.at[pl.ds(base, BPW)]
    pltpu.make_async_copy(my_out, my_out, sem).wait()


def kernel(harmful, emb_weight):
    idx = jnp.reshape(harmful.astype(jnp.int32), (NW, BPW))
    return _lookup(idx, emb_weight)
